# baseline (speedup 1.0000x reference)
import functools

import jax
import jax.numpy as jnp
from jax import lax
from jax.experimental import pallas as pl
from jax.experimental.pallas import tpu as pltpu
from jax.experimental.pallas import tpu_sc as plsc

N = 10000
E = 320000
IN_FEATS = 128
HIDDEN = 256

NC = 2
NS = 16
EPT = E // (NC * NS)
CHUNK = 80
NCH = EPT // CHUNK
ROWS = N
RPT = ROWS // NS
D1 = 144
D2 = 16


def _make_seg_sum(D, TD, split_out):
    d_lanes = D // 16
    mesh = plsc.VectorSubcoreMesh(core_axis_name="c", subcore_axis_name="s")
    if split_out:
        out_type = [jax.ShapeDtypeStruct((NC, ROWS, TD), jnp.float32),
                    jax.ShapeDtypeStruct((NC, ROWS, D - TD), jnp.float32)]
    else:
        out_type = [jax.ShapeDtypeStruct((NC, ROWS, D), jnp.float32)]

    @functools.partial(
        pl.kernel,
        mesh=mesh,
        compiler_params=pltpu.CompilerParams(use_tc_tiling_on_sc=False),
        out_type=out_type,
        scratch_types=[
            pltpu.VMEM((EPT,), jnp.int32),
            pltpu.VMEM((2, CHUNK), jnp.int32),
            pltpu.VMEM((2, CHUNK), jnp.int32),
            pltpu.VMEM((2, CHUNK, D), jnp.float32),
            pltpu.VMEM_SHARED((ROWS, D), jnp.float32),
            pltpu.SemaphoreType.DMA,
            pltpu.SemaphoreType.DMA,
            pltpu.SemaphoreType.DMA,
            pltpu.SemaphoreType.DMA,
        ],
    )
    def seg_sum(table_hbm, pk_hbm, *out_and_scratch):
        if split_out:
            out_hbm, out2_hbm = out_and_scratch[:2]
            rest = out_and_scratch[2:]
        else:
            out_hbm, = out_and_scratch[:1]
            rest = out_and_scratch[1:]
        pk, srcb, dstb, rows, acc, sem0, sem1, sem2, sem3 = rest
        c = lax.axis_index("c")
        s = lax.axis_index("s")
        sems = (sem0, sem1)
        ssems = (sem2, sem3)

        def zbody(t, carry):
            i = t // d_lanes
            j = t - i * d_lanes
            rows[0, i, pl.ds(j * 16, 16)] = jnp.zeros((16,), jnp.float32)
            return carry

        lax.fori_loop(0, CHUNK * d_lanes, zbody, 0)
        for z in range(RPT // CHUNK):
            pltpu.sync_copy(rows.at[0],
                            acc.at[pl.ds(s * RPT + z * CHUNK, CHUNK)])
        rem = RPT % CHUNK
        if rem:
            pltpu.sync_copy(
                rows.at[0, pl.ds(0, rem)],
                acc.at[pl.ds(s * RPT + (RPT // CHUNK) * CHUNK, rem)])

        base = (c * NS + s) * EPT
        pltpu.sync_copy(pk_hbm.at[pl.ds(base, EPT)], pk)

        def unpack(i, slot):
            for t in range(CHUNK // 16):
                v = pk[pl.ds(i * CHUNK + t * 16, 16)]
                srcb[slot, pl.ds(t * 16, 16)] = v & 16383
                dstb[slot, pl.ds(t * 16, 16)] = lax.shift_right_logical(v, 14)

        def gather_start(slot):
            pltpu.make_async_copy(
                table_hbm.at[srcb.at[slot]], rows.at[slot], sems[slot]).start()

        def gather_wait(slot):
            pltpu.make_async_copy(
                table_hbm.at[srcb.at[slot]], rows.at[slot], sems[slot]).wait()

        def scatter_start(slot):
            pltpu.async_copy(rows.at[slot], acc.at[dstb.at[slot]],
                             ssems[slot], add=True)

        def scatter_wait(slot):
            pltpu.make_async_copy(
                rows.at[slot], acc.at[dstb.at[slot]], ssems[slot]).wait()

        unpack(0, 0)
        gather_start(0)
        unpack(1, 1)
        gather_start(1)
        plsc.subcore_barrier()

        def body(j, carry):
            i0 = 2 * j
            gather_wait(0)
            scatter_start(0)
            gather_wait(1)
            scatter_start(1)

            scatter_wait(0)
            unpack(i0 + 2, 0)
            gather_start(0)

            @pl.when(j < NCH // 2 - 1)
            def _():
                scatter_wait(1)
                unpack(i0 + 3, 1)
                gather_start(1)

            return carry

        lax.fori_loop(0, NCH // 2, body, 0)
        gather_wait(0)
        scatter_start(0)
        scatter_wait(1)
        scatter_wait(0)

        plsc.subcore_barrier()
        if split_out:
            pltpu.sync_copy(acc.at[pl.ds(s * RPT, RPT), pl.ds(0, TD)],
                            out_hbm.at[c, pl.ds(s * RPT, RPT)])
            pltpu.sync_copy(acc.at[pl.ds(s * RPT, RPT), pl.ds(TD, D - TD)],
                            out2_hbm.at[c, pl.ds(s * RPT, RPT)])
        else:
            pltpu.sync_copy(acc.at[pl.ds(s * RPT, RPT)],
                            out_hbm.at[c, pl.ds(s * RPT, RPT)])

    return seg_sum


_seg_sum_d1 = _make_seg_sum(D1, IN_FEATS, True)
_seg_sum_d2 = _make_seg_sum(D2, D2, False)

_R = 2000


def _dense_body(x_ref, accf_ref, accd_ref, ws1_ref, wn1_ref, b1_ref,
                ws2_ref, wn2_ref, b2_ref, paug_ref, s_ref, rinv_ref):
    a = accf_ref[0] + accf_ref[1]
    ad = accd_ref[0] + accd_ref[1]
    deg = jnp.maximum(ad[:, 0:1], 1.0)
    hn = a / deg
    h = jnp.dot(x_ref[...], ws1_ref[...], preferred_element_type=jnp.float32)
    h = h + jnp.dot(hn, wn1_ref[...], preferred_element_type=jnp.float32)
    h = jnp.maximum(h + b1_ref[...], 0.0)
    paug_ref[...] = jnp.dot(h, wn2_ref[...], preferred_element_type=jnp.float32)
    s_ref[...] = (jnp.dot(h, ws2_ref[...], preferred_element_type=jnp.float32)
                  + b2_ref[...])
    rinv_ref[...] = 1.0 / deg


def _epilogue_body(s_ref, acc2_ref, rinv_ref, out_ref):
    a2 = acc2_ref[0] + acc2_ref[1]
    out_ref[...] = (s_ref[...] + a2 * rinv_ref[...])[:, :2]


def kernel(inputs, edge_index, W_self1, W_neigh1, b1, W_self2, W_neigh2, b2):
    x = inputs
    src = edge_index[0].astype(jnp.int32)
    dst = edge_index[1].astype(jnp.int32)
    pk = src | (dst << 14)

    xaug = jnp.concatenate(
        [x, jnp.ones((N, 1), x.dtype), jnp.zeros((N, D1 - IN_FEATS - 1),
                                                 x.dtype)], axis=1)
    accf, accd = _seg_sum_d1(xaug, pk)

    grid = (N // _R,)
    full = lambda shape: pl.BlockSpec(shape, lambda i: (0,) * len(shape))
    paug, s16, rinv = pl.pallas_call(
        _dense_body,
        grid=grid,
        in_specs=[
            pl.BlockSpec((_R, IN_FEATS), lambda i: (i, 0)),
            pl.BlockSpec((NC, _R, IN_FEATS), lambda i: (0, i, 0)),
            pl.BlockSpec((NC, _R, D1 - IN_FEATS), lambda i: (0, i, 0)),
            full((IN_FEATS, HIDDEN)),
            full((IN_FEATS, HIDDEN)),
            full((1, HIDDEN)),
            full((HIDDEN, D2)),
            full((HIDDEN, D2)),
            full((1, D2)),
        ],
        out_specs=[
            pl.BlockSpec((_R, D2), lambda i: (i, 0)),
            pl.BlockSpec((_R, D2), lambda i: (i, 0)),
            pl.BlockSpec((_R, 1), lambda i: (i, 0)),
        ],
        out_shape=[
            jax.ShapeDtypeStruct((N, D2), jnp.float32),
            jax.ShapeDtypeStruct((N, D2), jnp.float32),
            jax.ShapeDtypeStruct((N, 1), jnp.float32),
        ],
    )(x, accf, accd, W_self1, W_neigh1, b1.reshape(1, HIDDEN),
      jnp.pad(W_self2, ((0, 0), (0, D2 - 2))),
      jnp.pad(W_neigh2, ((0, 0), (0, D2 - 2))),
      jnp.pad(b2, (0, D2 - 2)).reshape(1, D2))

    (acc2,) = _seg_sum_d2(paug, pk)

    out = pl.pallas_call(
        _epilogue_body,
        grid=grid,
        in_specs=[
            pl.BlockSpec((_R, D2), lambda i: (i, 0)),
            pl.BlockSpec((NC, _R, D2), lambda i: (0, i, 0)),
            pl.BlockSpec((_R, 1), lambda i: (i, 0)),
        ],
        out_specs=pl.BlockSpec((_R, 2), lambda i: (i, 0)),
        out_shape=jax.ShapeDtypeStruct((N, 2), jnp.float32),
    )(s16, acc2, rinv)
    return out


# speedup vs baseline: 1.1516x; 1.1516x over previous
import functools

import jax
import jax.numpy as jnp
from jax import lax
from jax.experimental import pallas as pl
from jax.experimental.pallas import tpu as pltpu
from jax.experimental.pallas import tpu_sc as plsc

N = 10000
E = 320000
IN_FEATS = 128
HIDDEN = 256

NC = 2
NS = 16
EPT = E // (NC * NS)
CHUNK = 80
NCH = EPT // CHUNK
ROWS = N
RPT = ROWS // NS
D1 = 144
D2 = 16


def _make_seg_sum(D, TD, split_out):
    d_lanes = D // 16
    mesh = plsc.VectorSubcoreMesh(core_axis_name="c", subcore_axis_name="s")
    if split_out:
        out_type = [jax.ShapeDtypeStruct((NC, ROWS, TD), jnp.float32),
                    jax.ShapeDtypeStruct((NC, ROWS, D - TD), jnp.float32)]
    else:
        out_type = [jax.ShapeDtypeStruct((NC, ROWS, D), jnp.float32)]

    @functools.partial(
        pl.kernel,
        mesh=mesh,
        compiler_params=pltpu.CompilerParams(use_tc_tiling_on_sc=False),
        out_type=out_type,
        scratch_types=[
            pltpu.VMEM((EPT,), jnp.int32),
            pltpu.VMEM((2, CHUNK), jnp.int32),
            pltpu.VMEM((2, CHUNK), jnp.int32),
            pltpu.VMEM((2, CHUNK, D), jnp.float32),
            pltpu.VMEM_SHARED((ROWS, D), jnp.float32),
            pltpu.SemaphoreType.DMA,
            pltpu.SemaphoreType.DMA,
        ],
    )
    def seg_sum(table_hbm, pk_hbm, *out_and_scratch):
        if split_out:
            out_hbm, out2_hbm = out_and_scratch[:2]
            rest = out_and_scratch[2:]
        else:
            out_hbm, = out_and_scratch[:1]
            rest = out_and_scratch[1:]
        pk, srcb, dstb, rows, acc, sem0, sem1 = rest
        c = lax.axis_index("c")
        s = lax.axis_index("s")
        sems = (sem0, sem1)

        def zbody(t, carry):
            i = t // d_lanes
            j = t - i * d_lanes
            rows[0, i, pl.ds(j * 16, 16)] = jnp.zeros((16,), jnp.float32)
            return carry

        lax.fori_loop(0, CHUNK * d_lanes, zbody, 0)
        for z in range(RPT // CHUNK):
            pltpu.sync_copy(rows.at[0],
                            acc.at[pl.ds(s * RPT + z * CHUNK, CHUNK)])
        rem = RPT % CHUNK
        if rem:
            pltpu.sync_copy(
                rows.at[0, pl.ds(0, rem)],
                acc.at[pl.ds(s * RPT + (RPT // CHUNK) * CHUNK, rem)])

        base = (c * NS + s) * EPT
        pltpu.sync_copy(pk_hbm.at[pl.ds(base, EPT)], pk)

        def unpack(i, slot):
            for t in range(CHUNK // 16):
                v = pk[pl.ds(i * CHUNK + t * 16, 16)]
                srcb[slot, pl.ds(t * 16, 16)] = v & 16383
                dstb[slot, pl.ds(t * 16, 16)] = lax.shift_right_logical(v, 14)

        def gather_start(slot):
            pltpu.make_async_copy(
                table_hbm.at[srcb.at[slot]], rows.at[slot], sems[slot]).start()

        def gather_wait(slot):
            pltpu.make_async_copy(
                table_hbm.at[srcb.at[slot]], rows.at[slot], sems[slot]).wait()

        def scatter(slot):
            pltpu.sync_copy(rows.at[slot], acc.at[dstb.at[slot]], add=True)

        unpack(0, 0)
        gather_start(0)
        unpack(1, 1)
        gather_start(1)
        plsc.subcore_barrier()

        def body(j, carry):
            i0 = 2 * j
            gather_wait(0)
            scatter(0)
            unpack(i0 + 2, 0)
            gather_start(0)

            gather_wait(1)
            scatter(1)

            @pl.when(j < NCH // 2 - 1)
            def _():
                unpack(i0 + 3, 1)
                gather_start(1)

            return carry

        lax.fori_loop(0, NCH // 2, body, 0)
        gather_wait(0)
        scatter(0)

        plsc.subcore_barrier()
        if split_out:
            pltpu.sync_copy(acc.at[pl.ds(s * RPT, RPT), pl.ds(0, TD)],
                            out_hbm.at[c, pl.ds(s * RPT, RPT)])
            pltpu.sync_copy(acc.at[pl.ds(s * RPT, RPT), pl.ds(TD, D - TD)],
                            out2_hbm.at[c, pl.ds(s * RPT, RPT)])
        else:
            pltpu.sync_copy(acc.at[pl.ds(s * RPT, RPT)],
                            out_hbm.at[c, pl.ds(s * RPT, RPT)])

    return seg_sum


_seg_sum_d1 = _make_seg_sum(D1, IN_FEATS, True)
_seg_sum_d2 = _make_seg_sum(D2, D2, False)

_R = 2000


def _dense_body(x_ref, accf_ref, accd_ref, ws1_ref, wn1_ref, b1_ref,
                ws2_ref, wn2_ref, b2_ref, paug_ref, s_ref, rinv_ref):
    a = accf_ref[0] + accf_ref[1]
    ad = accd_ref[0] + accd_ref[1]
    deg = jnp.maximum(ad[:, 0:1], 1.0)
    hn = a / deg
    h = jnp.dot(x_ref[...], ws1_ref[...], preferred_element_type=jnp.float32)
    h = h + jnp.dot(hn, wn1_ref[...], preferred_element_type=jnp.float32)
    h = jnp.maximum(h + b1_ref[...], 0.0)
    paug_ref[...] = jnp.dot(h, wn2_ref[...], preferred_element_type=jnp.float32)
    s_ref[...] = (jnp.dot(h, ws2_ref[...], preferred_element_type=jnp.float32)
                  + b2_ref[...])
    rinv_ref[...] = 1.0 / deg


def _epilogue_body(s_ref, acc2_ref, rinv_ref, out_ref):
    a2 = acc2_ref[0] + acc2_ref[1]
    out_ref[...] = (s_ref[...] + a2 * rinv_ref[...])[:, :2]


def kernel(inputs, edge_index, W_self1, W_neigh1, b1, W_self2, W_neigh2, b2):
    x = inputs
    src = edge_index[0].astype(jnp.int32)
    dst = edge_index[1].astype(jnp.int32)
    pk = src | (dst << 14)

    xaug = jnp.concatenate(
        [x, jnp.ones((N, 1), x.dtype), jnp.zeros((N, D1 - IN_FEATS - 1),
                                                 x.dtype)], axis=1)
    accf, accd = _seg_sum_d1(xaug, pk)

    grid = (N // _R,)
    full = lambda shape: pl.BlockSpec(shape, lambda i: (0,) * len(shape))
    paug, s16, rinv = pl.pallas_call(
        _dense_body,
        grid=grid,
        in_specs=[
            pl.BlockSpec((_R, IN_FEATS), lambda i: (i, 0)),
            pl.BlockSpec((NC, _R, IN_FEATS), lambda i: (0, i, 0)),
            pl.BlockSpec((NC, _R, D1 - IN_FEATS), lambda i: (0, i, 0)),
            full((IN_FEATS, HIDDEN)),
            full((IN_FEATS, HIDDEN)),
            full((1, HIDDEN)),
            full((HIDDEN, D2)),
            full((HIDDEN, D2)),
            full((1, D2)),
        ],
        out_specs=[
            pl.BlockSpec((_R, D2), lambda i: (i, 0)),
            pl.BlockSpec((_R, D2), lambda i: (i, 0)),
            pl.BlockSpec((_R, 1), lambda i: (i, 0)),
        ],
        out_shape=[
            jax.ShapeDtypeStruct((N, D2), jnp.float32),
            jax.ShapeDtypeStruct((N, D2), jnp.float32),
            jax.ShapeDtypeStruct((N, 1), jnp.float32),
        ],
    )(x, accf, accd, W_self1, W_neigh1, b1.reshape(1, HIDDEN),
      jnp.pad(W_self2, ((0, 0), (0, D2 - 2))),
      jnp.pad(W_neigh2, ((0, 0), (0, D2 - 2))),
      jnp.pad(b2, (0, D2 - 2)).reshape(1, D2))

    (acc2,) = _seg_sum_d2(paug, pk)

    out = pl.pallas_call(
        _epilogue_body,
        grid=grid,
        in_specs=[
            pl.BlockSpec((_R, D2), lambda i: (i, 0)),
            pl.BlockSpec((NC, _R, D2), lambda i: (0, i, 0)),
            pl.BlockSpec((_R, 1), lambda i: (i, 0)),
        ],
        out_specs=pl.BlockSpec((_R, 2), lambda i: (i, 0)),
        out_shape=jax.ShapeDtypeStruct((N, 2), jnp.float32),
    )(s16, acc2, rinv)
    return out


# speedup vs baseline: 1.2045x; 1.0460x over previous
import functools

import jax
import jax.numpy as jnp
from jax import lax
from jax.experimental import pallas as pl
from jax.experimental.pallas import tpu as pltpu
from jax.experimental.pallas import tpu_sc as plsc

N = 10000
E = 320000
IN_FEATS = 128
HIDDEN = 256

NC = 2
NS = 16
EPT = E // (NC * NS)
CHUNK = 80
NCH = EPT // CHUNK
ROWS = N
RPT = ROWS // NS
D1 = 144
D2 = 16


def _make_seg_sum(D, TD, split_out):
    d_lanes = D // 16
    mesh = plsc.VectorSubcoreMesh(core_axis_name="c", subcore_axis_name="s")
    if split_out:
        out_type = [jax.ShapeDtypeStruct((NC, ROWS, TD), jnp.float32),
                    jax.ShapeDtypeStruct((NC, ROWS, D - TD), jnp.float32)]
    else:
        out_type = [jax.ShapeDtypeStruct((NC, ROWS, D), jnp.float32)]

    @functools.partial(
        pl.kernel,
        mesh=mesh,
        compiler_params=pltpu.CompilerParams(use_tc_tiling_on_sc=False),
        out_type=out_type,
        scratch_types=[
            pltpu.VMEM((EPT,), jnp.int32),
            pltpu.VMEM((2, CHUNK), jnp.int32),
            pltpu.VMEM((2, CHUNK), jnp.int32),
            pltpu.VMEM((2, CHUNK, D), jnp.float32),
            pltpu.VMEM_SHARED((ROWS, D), jnp.float32),
            pltpu.SemaphoreType.DMA,
            pltpu.SemaphoreType.DMA,
        ],
    )
    def seg_sum(table_hbm, pk_hbm, *out_and_scratch):
        if split_out:
            out_hbm, out2_hbm = out_and_scratch[:2]
            rest = out_and_scratch[2:]
        else:
            out_hbm, = out_and_scratch[:1]
            rest = out_and_scratch[1:]
        pk, srcb, dstb, rows, acc, sem0, sem1 = rest
        c = lax.axis_index("c")
        s = lax.axis_index("s")
        sems = (sem0, sem1)

        def zbody(t, carry):
            i = t // d_lanes
            j = t - i * d_lanes
            rows[0, i, pl.ds(j * 16, 16)] = jnp.zeros((16,), jnp.float32)
            return carry

        lax.fori_loop(0, CHUNK * d_lanes, zbody, 0)
        for z in range(RPT // CHUNK):
            pltpu.sync_copy(rows.at[0],
                            acc.at[pl.ds(s * RPT + z * CHUNK, CHUNK)])
        rem = RPT % CHUNK
        if rem:
            pltpu.sync_copy(
                rows.at[0, pl.ds(0, rem)],
                acc.at[pl.ds(s * RPT + (RPT // CHUNK) * CHUNK, rem)])

        base = (c * NS + s) * EPT
        pltpu.sync_copy(pk_hbm.at[pl.ds(base, EPT)], pk)

        def unpack(i, slot):
            for t in range(CHUNK // 16):
                v = pk[pl.ds(i * CHUNK + t * 16, 16)]
                srcb[slot, pl.ds(t * 16, 16)] = v & 16383
                dstb[slot, pl.ds(t * 16, 16)] = lax.shift_right_logical(v, 14)

        def gather_start(slot):
            pltpu.make_async_copy(
                table_hbm.at[srcb.at[slot]], rows.at[slot], sems[slot]).start()

        def gather_wait(slot):
            pltpu.make_async_copy(
                table_hbm.at[srcb.at[slot]], rows.at[slot], sems[slot]).wait()

        def scatter(slot):
            pltpu.sync_copy(rows.at[slot], acc.at[dstb.at[slot]], add=True)

        unpack(0, 0)
        gather_start(0)
        unpack(1, 1)
        gather_start(1)
        plsc.subcore_barrier()

        def body(j, carry):
            i0 = 2 * j
            gather_wait(0)
            scatter(0)
            unpack(i0 + 2, 0)
            gather_start(0)

            gather_wait(1)
            scatter(1)

            @pl.when(j < NCH // 2 - 1)
            def _():
                unpack(i0 + 3, 1)
                gather_start(1)

            return carry

        lax.fori_loop(0, NCH // 2, body, 0)
        gather_wait(0)
        scatter(0)

        plsc.subcore_barrier()
        if split_out:
            pltpu.sync_copy(acc.at[pl.ds(s * RPT, RPT), pl.ds(0, TD)],
                            out_hbm.at[c, pl.ds(s * RPT, RPT)])
            pltpu.sync_copy(acc.at[pl.ds(s * RPT, RPT), pl.ds(TD, D - TD)],
                            out2_hbm.at[c, pl.ds(s * RPT, RPT)])
        else:
            pltpu.sync_copy(acc.at[pl.ds(s * RPT, RPT)],
                            out_hbm.at[c, pl.ds(s * RPT, RPT)])

    return seg_sum


_seg_sum_d1 = _make_seg_sum(D1, IN_FEATS, True)
_seg_sum_d2 = _make_seg_sum(D2, D2, False)

_R = 2000


def _dense_body(x_ref, accf_ref, accd_ref, ws1_ref, wn1_ref, b1_ref,
                ws2_ref, wn2_ref, b2_ref, paug_ref, s_ref, rinv_ref):
    a = accf_ref[0] + accf_ref[1]
    ad = accd_ref[0] + accd_ref[1]
    deg = jnp.maximum(ad[:, 0:1], 1.0)
    hn = a / deg
    h = jnp.dot(x_ref[...], ws1_ref[...], preferred_element_type=jnp.float32)
    h = h + jnp.dot(hn, wn1_ref[...], preferred_element_type=jnp.float32)
    h = jnp.maximum(h + b1_ref[...], 0.0)
    paug_ref[...] = jnp.dot(h, wn2_ref[...], preferred_element_type=jnp.float32)
    s_ref[...] = (jnp.dot(h, ws2_ref[...], preferred_element_type=jnp.float32)
                  + b2_ref[...])
    rinv_ref[...] = 1.0 / deg


def _pack_body(e_ref, out_ref):
    out_ref[...] = e_ref[0] | (e_ref[1] << 14)


def _epilogue_body(s_ref, acc2_ref, rinv_ref, out_ref):
    a2 = acc2_ref[0] + acc2_ref[1]
    out_ref[...] = (s_ref[...] + a2 * rinv_ref[...])[:, :2]


def kernel(inputs, edge_index, W_self1, W_neigh1, b1, W_self2, W_neigh2, b2):
    x = inputs
    ei = edge_index.astype(jnp.int32)
    pk = pl.pallas_call(
        _pack_body,
        grid=(1,),
        in_specs=[pl.BlockSpec((2, E), lambda i: (0, 0))],
        out_specs=pl.BlockSpec((E,), lambda i: (0,)),
        out_shape=jax.ShapeDtypeStruct((E,), jnp.int32),
    )(ei)

    xaug = jnp.concatenate(
        [x, jnp.ones((N, 1), x.dtype), jnp.zeros((N, D1 - IN_FEATS - 1),
                                                 x.dtype)], axis=1)
    accf, accd = _seg_sum_d1(xaug, pk)

    grid = (N // _R,)
    full = lambda shape: pl.BlockSpec(shape, lambda i: (0,) * len(shape))
    paug, s16, rinv = pl.pallas_call(
        _dense_body,
        grid=grid,
        in_specs=[
            pl.BlockSpec((_R, IN_FEATS), lambda i: (i, 0)),
            pl.BlockSpec((NC, _R, IN_FEATS), lambda i: (0, i, 0)),
            pl.BlockSpec((NC, _R, D1 - IN_FEATS), lambda i: (0, i, 0)),
            full((IN_FEATS, HIDDEN)),
            full((IN_FEATS, HIDDEN)),
            full((1, HIDDEN)),
            full((HIDDEN, D2)),
            full((HIDDEN, D2)),
            full((1, D2)),
        ],
        out_specs=[
            pl.BlockSpec((_R, D2), lambda i: (i, 0)),
            pl.BlockSpec((_R, D2), lambda i: (i, 0)),
            pl.BlockSpec((_R, 1), lambda i: (i, 0)),
        ],
        out_shape=[
            jax.ShapeDtypeStruct((N, D2), jnp.float32),
            jax.ShapeDtypeStruct((N, D2), jnp.float32),
            jax.ShapeDtypeStruct((N, 1), jnp.float32),
        ],
    )(x, accf, accd, W_self1, W_neigh1, b1.reshape(1, HIDDEN),
      jnp.pad(W_self2, ((0, 0), (0, D2 - 2))),
      jnp.pad(W_neigh2, ((0, 0), (0, D2 - 2))),
      jnp.pad(b2, (0, D2 - 2)).reshape(1, D2))

    (acc2,) = _seg_sum_d2(paug, pk)

    out = pl.pallas_call(
        _epilogue_body,
        grid=grid,
        in_specs=[
            pl.BlockSpec((_R, D2), lambda i: (i, 0)),
            pl.BlockSpec((NC, _R, D2), lambda i: (0, i, 0)),
            pl.BlockSpec((_R, 1), lambda i: (i, 0)),
        ],
        out_specs=pl.BlockSpec((_R, 2), lambda i: (i, 0)),
        out_shape=jax.ShapeDtypeStruct((N, 2), jnp.float32),
    )(s16, acc2, rinv)
    return out


# speedup vs baseline: 1.2113x; 1.0056x over previous
import functools

import jax
import jax.numpy as jnp
from jax import lax
from jax.experimental import pallas as pl
from jax.experimental.pallas import tpu as pltpu
from jax.experimental.pallas import tpu_sc as plsc

N = 10000
E = 320000
IN_FEATS = 128
HIDDEN = 256

NC = 2
NS = 16
EPT = E // (NC * NS)
CHUNK = 80
NCH = EPT // CHUNK
ROWS = N
RPT = ROWS // NS
D1 = 144
D2 = 16


def _make_seg_sum(D, TD, split_out):
    d_lanes = D // 16
    mesh = plsc.VectorSubcoreMesh(core_axis_name="c", subcore_axis_name="s")
    if split_out:
        out_type = [jax.ShapeDtypeStruct((NC, ROWS, TD), jnp.float32),
                    jax.ShapeDtypeStruct((NC, ROWS, D - TD), jnp.float32)]
    else:
        out_type = [jax.ShapeDtypeStruct((NC, ROWS, D), jnp.float32)]

    @functools.partial(
        pl.kernel,
        mesh=mesh,
        compiler_params=pltpu.CompilerParams(use_tc_tiling_on_sc=False),
        out_type=out_type,
        scratch_types=[
            pltpu.VMEM((EPT,), jnp.int32),
            pltpu.VMEM((2, CHUNK), jnp.int32),
            pltpu.VMEM((2, CHUNK), jnp.int32),
            pltpu.VMEM((2, CHUNK, D), jnp.float32),
            pltpu.VMEM_SHARED((ROWS, D), jnp.float32),
            pltpu.SemaphoreType.DMA,
            pltpu.SemaphoreType.DMA,
        ],
    )
    def seg_sum(table_hbm, pk_hbm, *out_and_scratch):
        if split_out:
            out_hbm, out2_hbm = out_and_scratch[:2]
            rest = out_and_scratch[2:]
        else:
            out_hbm, = out_and_scratch[:1]
            rest = out_and_scratch[1:]
        pk, srcb, dstb, rows, acc, sem0, sem1 = rest
        c = lax.axis_index("c")
        s = lax.axis_index("s")
        sems = (sem0, sem1)

        def zbody(t, carry):
            i = t // d_lanes
            j = t - i * d_lanes
            rows[0, i, pl.ds(j * 16, 16)] = jnp.zeros((16,), jnp.float32)
            return carry

        lax.fori_loop(0, CHUNK * d_lanes, zbody, 0)
        for z in range(RPT // CHUNK):
            pltpu.sync_copy(rows.at[0],
                            acc.at[pl.ds(s * RPT + z * CHUNK, CHUNK)])
        rem = RPT % CHUNK
        if rem:
            pltpu.sync_copy(
                rows.at[0, pl.ds(0, rem)],
                acc.at[pl.ds(s * RPT + (RPT // CHUNK) * CHUNK, rem)])

        base = (c * NS + s) * EPT
        pltpu.sync_copy(pk_hbm.at[pl.ds(base, EPT)], pk)

        def unpack(i, slot):
            for t in range(CHUNK // 16):
                v = pk[pl.ds(i * CHUNK + t * 16, 16)]
                srcb[slot, pl.ds(t * 16, 16)] = v & 16383
                dstb[slot, pl.ds(t * 16, 16)] = lax.shift_right_logical(v, 14)

        def gather_start(slot):
            pltpu.make_async_copy(
                table_hbm.at[srcb.at[slot]], rows.at[slot], sems[slot]).start()

        def gather_wait(slot):
            pltpu.make_async_copy(
                table_hbm.at[srcb.at[slot]], rows.at[slot], sems[slot]).wait()

        def scatter(slot):
            pltpu.sync_copy(rows.at[slot], acc.at[dstb.at[slot]], add=True)

        unpack(0, 0)
        gather_start(0)
        unpack(1, 1)
        gather_start(1)
        plsc.subcore_barrier()

        def body(j, carry):
            i0 = 2 * j
            gather_wait(0)
            scatter(0)
            unpack(i0 + 2, 0)
            gather_start(0)

            gather_wait(1)
            scatter(1)

            @pl.when(j < NCH // 2 - 1)
            def _():
                unpack(i0 + 3, 1)
                gather_start(1)

            return carry

        lax.fori_loop(0, NCH // 2, body, 0)
        gather_wait(0)
        scatter(0)

        plsc.subcore_barrier()
        if split_out:
            pltpu.sync_copy(acc.at[pl.ds(s * RPT, RPT), pl.ds(0, TD)],
                            out_hbm.at[c, pl.ds(s * RPT, RPT)])
            pltpu.sync_copy(acc.at[pl.ds(s * RPT, RPT), pl.ds(TD, D - TD)],
                            out2_hbm.at[c, pl.ds(s * RPT, RPT)])
        else:
            pltpu.sync_copy(acc.at[pl.ds(s * RPT, RPT)],
                            out_hbm.at[c, pl.ds(s * RPT, RPT)])

    return seg_sum


_seg_sum_d1 = _make_seg_sum(D1, IN_FEATS, True)
_seg_sum_d2 = _make_seg_sum(D2, D2, False)

_R = 2000


def _dense_body(x_ref, accf_ref, accd_ref, ws1_ref, wn1_ref, b1_ref,
                w2_ref, b2_ref, paug_ref, s_ref, rinv_ref):
    a = accf_ref[0] + accf_ref[1]
    ad = accd_ref[0] + accd_ref[1]
    deg = jnp.maximum(ad[:, 0:1], 1.0)
    hn = a / deg
    h = jnp.dot(x_ref[...], ws1_ref[...], preferred_element_type=jnp.float32)
    h = h + jnp.dot(hn, wn1_ref[...], preferred_element_type=jnp.float32)
    h = jnp.maximum(h + b1_ref[...], 0.0)
    ps = jnp.dot(h, w2_ref[...], preferred_element_type=jnp.float32)
    paug_ref[...] = ps[:, :D2]
    s_ref[...] = ps[:, D2:] + b2_ref[...]
    rinv_ref[...] = 1.0 / deg


def _pack_body(e_ref, out_ref):
    out_ref[...] = e_ref[0] | (e_ref[1] << 14)


def _epilogue_body(s_ref, acc2_ref, rinv_ref, out_ref):
    a2 = acc2_ref[0] + acc2_ref[1]
    out_ref[...] = (s_ref[...] + a2 * rinv_ref[...])[:, :2]


def kernel(inputs, edge_index, W_self1, W_neigh1, b1, W_self2, W_neigh2, b2):
    x = inputs
    ei = edge_index.astype(jnp.int32)
    pk = pl.pallas_call(
        _pack_body,
        grid=(1,),
        in_specs=[pl.BlockSpec((2, E), lambda i: (0, 0))],
        out_specs=pl.BlockSpec((E,), lambda i: (0,)),
        out_shape=jax.ShapeDtypeStruct((E,), jnp.int32),
    )(ei)

    xaug = jnp.concatenate(
        [x, jnp.ones((N, 1), x.dtype), jnp.zeros((N, D1 - IN_FEATS - 1),
                                                 x.dtype)], axis=1)
    accf, accd = _seg_sum_d1(xaug, pk)

    grid = (N // _R,)
    full = lambda shape: pl.BlockSpec(shape, lambda i: (0,) * len(shape))
    paug, s16, rinv = pl.pallas_call(
        _dense_body,
        grid=grid,
        in_specs=[
            pl.BlockSpec((_R, IN_FEATS), lambda i: (i, 0)),
            pl.BlockSpec((NC, _R, IN_FEATS), lambda i: (0, i, 0)),
            pl.BlockSpec((NC, _R, D1 - IN_FEATS), lambda i: (0, i, 0)),
            full((IN_FEATS, HIDDEN)),
            full((IN_FEATS, HIDDEN)),
            full((1, HIDDEN)),
            full((HIDDEN, 2 * D2)),
            full((1, D2)),
        ],
        out_specs=[
            pl.BlockSpec((_R, D2), lambda i: (i, 0)),
            pl.BlockSpec((_R, D2), lambda i: (i, 0)),
            pl.BlockSpec((_R, 1), lambda i: (i, 0)),
        ],
        out_shape=[
            jax.ShapeDtypeStruct((N, D2), jnp.float32),
            jax.ShapeDtypeStruct((N, D2), jnp.float32),
            jax.ShapeDtypeStruct((N, 1), jnp.float32),
        ],
    )(x, accf, accd, W_self1, W_neigh1, b1.reshape(1, HIDDEN),
      jnp.concatenate([jnp.pad(W_neigh2, ((0, 0), (0, D2 - 2))),
                       jnp.pad(W_self2, ((0, 0), (0, D2 - 2)))], axis=1),
      jnp.pad(b2, (0, D2 - 2)).reshape(1, D2))

    (acc2,) = _seg_sum_d2(paug, pk)

    out = pl.pallas_call(
        _epilogue_body,
        grid=grid,
        in_specs=[
            pl.BlockSpec((_R, D2), lambda i: (i, 0)),
            pl.BlockSpec((NC, _R, D2), lambda i: (0, i, 0)),
            pl.BlockSpec((_R, 1), lambda i: (i, 0)),
        ],
        out_specs=pl.BlockSpec((_R, 2), lambda i: (i, 0)),
        out_shape=jax.ShapeDtypeStruct((N, 2), jnp.float32),
    )(s16, acc2, rinv)
    return out


# speedup vs baseline: 1.3226x; 1.0919x over previous
import functools

import jax
import jax.numpy as jnp
from jax import lax
from jax.experimental import pallas as pl
from jax.experimental.pallas import tpu as pltpu
from jax.experimental.pallas import tpu_sc as plsc

N = 10000
E = 320000
IN_FEATS = 128
HIDDEN = 256

NC = 2
NS = 16
EPT = E // (NC * NS)
CHUNK = 80
NCH = EPT // CHUNK
ROWS = N
RPT = ROWS // NS
D1 = 144
D2 = 16


def _make_seg_sum(D, TD, split_out, NSLOT=2):
    d_lanes = D // 16
    mesh = plsc.VectorSubcoreMesh(core_axis_name="c", subcore_axis_name="s")
    if split_out:
        out_type = [jax.ShapeDtypeStruct((NC, ROWS, TD), jnp.float32),
                    jax.ShapeDtypeStruct((NC, ROWS, D - TD), jnp.float32)]
    else:
        out_type = [jax.ShapeDtypeStruct((NC, ROWS, D), jnp.float32)]

    @functools.partial(
        pl.kernel,
        mesh=mesh,
        compiler_params=pltpu.CompilerParams(use_tc_tiling_on_sc=False),
        out_type=out_type,
        scratch_types=[
            pltpu.VMEM((EPT,), jnp.int32),
            pltpu.VMEM((NSLOT, CHUNK), jnp.int32),
            pltpu.VMEM((NSLOT, CHUNK), jnp.int32),
            pltpu.VMEM((NSLOT, CHUNK, D), jnp.float32),
            pltpu.VMEM_SHARED((ROWS, D), jnp.float32),
        ] + [pltpu.SemaphoreType.DMA] * NSLOT,
    )
    def seg_sum(table_hbm, pk_hbm, *out_and_scratch):
        if split_out:
            out_hbm, out2_hbm = out_and_scratch[:2]
            rest = out_and_scratch[2:]
        else:
            out_hbm, = out_and_scratch[:1]
            rest = out_and_scratch[1:]
        pk, srcb, dstb, rows, acc = rest[:5]
        sems = tuple(rest[5:])
        c = lax.axis_index("c")
        s = lax.axis_index("s")

        def zbody(t, carry):
            i = t // d_lanes
            j = t - i * d_lanes
            rows[0, i, pl.ds(j * 16, 16)] = jnp.zeros((16,), jnp.float32)
            return carry

        lax.fori_loop(0, CHUNK * d_lanes, zbody, 0)
        for z in range(RPT // CHUNK):
            pltpu.sync_copy(rows.at[0],
                            acc.at[pl.ds(s * RPT + z * CHUNK, CHUNK)])
        rem = RPT % CHUNK
        if rem:
            pltpu.sync_copy(
                rows.at[0, pl.ds(0, rem)],
                acc.at[pl.ds(s * RPT + (RPT // CHUNK) * CHUNK, rem)])

        base = (c * NS + s) * EPT
        pltpu.sync_copy(pk_hbm.at[pl.ds(base, EPT)], pk)

        def unpack(i, slot):
            for t in range(CHUNK // 16):
                v = pk[pl.ds(i * CHUNK + t * 16, 16)]
                srcb[slot, pl.ds(t * 16, 16)] = v & 16383
                dstb[slot, pl.ds(t * 16, 16)] = lax.shift_right_logical(v, 14)

        def gather_start(slot):
            pltpu.make_async_copy(
                table_hbm.at[srcb.at[slot]], rows.at[slot], sems[slot]).start()

        def gather_wait(slot):
            pltpu.make_async_copy(
                table_hbm.at[srcb.at[slot]], rows.at[slot], sems[slot]).wait()

        def scatter(slot):
            pltpu.sync_copy(rows.at[slot], acc.at[dstb.at[slot]], add=True)

        for slot in range(NSLOT):
            unpack(slot, slot)
            gather_start(slot)
        plsc.subcore_barrier()

        def body(j, carry):
            for slot in range(NSLOT):
                i = NSLOT * j + slot
                gather_wait(slot)
                scatter(slot)

                @pl.when(i + NSLOT < NCH)
                def _():
                    unpack(i + NSLOT, slot)
                    gather_start(slot)

            return carry

        lax.fori_loop(0, NCH // NSLOT, body, 0)
        for r in range(NCH % NSLOT):
            gather_wait(r)
            scatter(r)

        plsc.subcore_barrier()
        if split_out:
            pltpu.sync_copy(acc.at[pl.ds(s * RPT, RPT), pl.ds(0, TD)],
                            out_hbm.at[c, pl.ds(s * RPT, RPT)])
            pltpu.sync_copy(acc.at[pl.ds(s * RPT, RPT), pl.ds(TD, D - TD)],
                            out2_hbm.at[c, pl.ds(s * RPT, RPT)])
        else:
            pltpu.sync_copy(acc.at[pl.ds(s * RPT, RPT)],
                            out_hbm.at[c, pl.ds(s * RPT, RPT)])

    return seg_sum


_seg_sum_d1 = _make_seg_sum(D1, IN_FEATS, True, NSLOT=2)
_seg_sum_d2 = _make_seg_sum(D2, D2, False, NSLOT=4)

_R = 2000


def _dense_body(x_ref, accf_ref, accd_ref, ws1_ref, wn1_ref, b1_ref,
                w2_ref, b2_ref, paug_ref, s_ref, rinv_ref):
    a = accf_ref[0] + accf_ref[1]
    ad = accd_ref[0] + accd_ref[1]
    deg = jnp.maximum(ad[:, 0:1], 1.0)
    hn = a / deg
    h = jnp.dot(x_ref[...], ws1_ref[...], preferred_element_type=jnp.float32)
    h = h + jnp.dot(hn, wn1_ref[...], preferred_element_type=jnp.float32)
    h = jnp.maximum(h + b1_ref[...], 0.0)
    ps = jnp.dot(h, w2_ref[...], preferred_element_type=jnp.float32)
    paug_ref[...] = ps[:, :D2]
    s_ref[...] = ps[:, D2:] + b2_ref[...]
    rinv_ref[...] = 1.0 / deg


def _pack_body(e_ref, out_ref):
    out_ref[...] = e_ref[0] | (e_ref[1] << 14)


def _epilogue_body(s_ref, acc2_ref, rinv_ref, out_ref):
    a2 = acc2_ref[0] + acc2_ref[1]
    out_ref[...] = (s_ref[...] + a2 * rinv_ref[...])[:, :2]


def kernel(inputs, edge_index, W_self1, W_neigh1, b1, W_self2, W_neigh2, b2):
    x = inputs
    ei = edge_index.astype(jnp.int32)
    pk = pl.pallas_call(
        _pack_body,
        grid=(1,),
        in_specs=[pl.BlockSpec((2, E), lambda i: (0, 0))],
        out_specs=pl.BlockSpec((E,), lambda i: (0,)),
        out_shape=jax.ShapeDtypeStruct((E,), jnp.int32),
    )(ei)

    xaug = jnp.concatenate(
        [x, jnp.ones((N, 1), x.dtype), jnp.zeros((N, D1 - IN_FEATS - 1),
                                                 x.dtype)], axis=1)
    accf, accd = _seg_sum_d1(xaug, pk)

    grid = (N // _R,)
    full = lambda shape: pl.BlockSpec(shape, lambda i: (0,) * len(shape))
    paug, s16, rinv = pl.pallas_call(
        _dense_body,
        grid=grid,
        in_specs=[
            pl.BlockSpec((_R, IN_FEATS), lambda i: (i, 0)),
            pl.BlockSpec((NC, _R, IN_FEATS), lambda i: (0, i, 0)),
            pl.BlockSpec((NC, _R, D1 - IN_FEATS), lambda i: (0, i, 0)),
            full((IN_FEATS, HIDDEN)),
            full((IN_FEATS, HIDDEN)),
            full((1, HIDDEN)),
            full((HIDDEN, 2 * D2)),
            full((1, D2)),
        ],
        out_specs=[
            pl.BlockSpec((_R, D2), lambda i: (i, 0)),
            pl.BlockSpec((_R, D2), lambda i: (i, 0)),
            pl.BlockSpec((_R, 1), lambda i: (i, 0)),
        ],
        out_shape=[
            jax.ShapeDtypeStruct((N, D2), jnp.float32),
            jax.ShapeDtypeStruct((N, D2), jnp.float32),
            jax.ShapeDtypeStruct((N, 1), jnp.float32),
        ],
    )(x, accf, accd, W_self1, W_neigh1, b1.reshape(1, HIDDEN),
      jnp.concatenate([jnp.pad(W_neigh2, ((0, 0), (0, D2 - 2))),
                       jnp.pad(W_self2, ((0, 0), (0, D2 - 2)))], axis=1),
      jnp.pad(b2, (0, D2 - 2)).reshape(1, D2))

    (acc2,) = _seg_sum_d2(paug, pk)

    out = pl.pallas_call(
        _epilogue_body,
        grid=grid,
        in_specs=[
            pl.BlockSpec((_R, D2), lambda i: (i, 0)),
            pl.BlockSpec((NC, _R, D2), lambda i: (0, i, 0)),
            pl.BlockSpec((_R, 1), lambda i: (i, 0)),
        ],
        out_specs=pl.BlockSpec((_R, 2), lambda i: (i, 0)),
        out_shape=jax.ShapeDtypeStruct((N, 2), jnp.float32),
    )(s16, acc2, rinv)
    return out


# speedup vs baseline: 1.4055x; 1.0627x over previous
import functools

import jax
import jax.numpy as jnp
from jax import lax
from jax.experimental import pallas as pl
from jax.experimental.pallas import tpu as pltpu
from jax.experimental.pallas import tpu_sc as plsc

N = 10000
E = 320000
IN_FEATS = 128
HIDDEN = 256

NC = 2
NS = 16
EPT = E // (NC * NS)
CHUNK = 80
NCH = EPT // CHUNK
ROWS = N
RPT = ROWS // NS
D1 = 144
D2 = 16


def _make_seg_sum(D, TD, split_out, NSLOT=2, WIN=None):
    d_lanes = D // 16
    mesh = plsc.VectorSubcoreMesh(core_axis_name="c", subcore_axis_name="s")
    if split_out:
        out_type = [jax.ShapeDtypeStruct((NC, ROWS, TD), jnp.float32),
                    jax.ShapeDtypeStruct((NC, ROWS, D - TD), jnp.float32)]
    else:
        out_type = [jax.ShapeDtypeStruct((NC, ROWS, D), jnp.float32)]

    @functools.partial(
        pl.kernel,
        mesh=mesh,
        compiler_params=pltpu.CompilerParams(use_tc_tiling_on_sc=False),
        out_type=out_type,
        scratch_types=[
            pltpu.VMEM((EPT,) if WIN is None else (2, WIN * CHUNK),
                       jnp.int32),
            pltpu.VMEM((NSLOT, CHUNK), jnp.int32),
            pltpu.VMEM((NSLOT, CHUNK), jnp.int32),
            pltpu.VMEM((NSLOT, CHUNK, D), jnp.float32),
            pltpu.VMEM_SHARED((ROWS, D), jnp.float32),
        ] + [pltpu.SemaphoreType.DMA] * (NSLOT + 1),
    )
    def seg_sum(table_hbm, pk_hbm, *out_and_scratch):
        if split_out:
            out_hbm, out2_hbm = out_and_scratch[:2]
            rest = out_and_scratch[2:]
        else:
            out_hbm, = out_and_scratch[:1]
            rest = out_and_scratch[1:]
        pk, srcb, dstb, rows, acc = rest[:5]
        sems = tuple(rest[5:5 + NSLOT])
        sem_pk = rest[5 + NSLOT]
        c = lax.axis_index("c")
        s = lax.axis_index("s")

        def zbody(t, carry):
            i = t // d_lanes
            j = t - i * d_lanes
            rows[0, i, pl.ds(j * 16, 16)] = jnp.zeros((16,), jnp.float32)
            return carry

        lax.fori_loop(0, CHUNK * d_lanes, zbody, 0)
        for z in range(RPT // CHUNK):
            pltpu.sync_copy(rows.at[0],
                            acc.at[pl.ds(s * RPT + z * CHUNK, CHUNK)])
        rem = RPT % CHUNK
        if rem:
            pltpu.sync_copy(
                rows.at[0, pl.ds(0, rem)],
                acc.at[pl.ds(s * RPT + (RPT // CHUNK) * CHUNK, rem)])

        base = (c * NS + s) * EPT

        def unpack_from(pkbuf, i, slot):
            for t in range(CHUNK // 16):
                v = pkbuf[pl.ds(i * CHUNK + t * 16, 16)]
                srcb[slot, pl.ds(t * 16, 16)] = v & 16383
                dstb[slot, pl.ds(t * 16, 16)] = lax.shift_right_logical(v, 14)

        def gather_start(slot):
            pltpu.make_async_copy(
                table_hbm.at[srcb.at[slot]], rows.at[slot], sems[slot]).start()

        def gather_wait(slot):
            pltpu.make_async_copy(
                table_hbm.at[srcb.at[slot]], rows.at[slot], sems[slot]).wait()

        def scatter(slot):
            pltpu.sync_copy(rows.at[slot], acc.at[dstb.at[slot]], add=True)

        def run_chunks(pkbuf, nch):
            def body(j, carry):
                for slot in range(NSLOT):
                    i = NSLOT * j + slot
                    gather_wait(slot)
                    scatter(slot)

                    @pl.when(i + NSLOT < nch)
                    def _():
                        unpack_from(pkbuf, i + NSLOT, slot)
                        gather_start(slot)

                return carry

            lax.fori_loop(0, nch // NSLOT, body, 0)
            for r in range(nch % NSLOT):
                gather_wait(r)
                scatter(r)

        if WIN is None:
            pltpu.sync_copy(pk_hbm.at[pl.ds(base, EPT)], pk)
            for slot in range(NSLOT):
                unpack_from(pk, slot, slot)
                gather_start(slot)
            plsc.subcore_barrier()
            run_chunks(pk, NCH)
        else:
            winw = WIN * CHUNK
            nwin = NCH // WIN

            def stage(w):
                return pltpu.make_async_copy(
                    pk_hbm.at[pl.ds(base + w * winw, winw)],
                    pk.at[w % 2], sem_pk)

            stage(0).start()
            plsc.subcore_barrier()
            stage(0).wait()
            for w in range(nwin):
                if w + 1 < nwin:
                    stage(w + 1).start()
                pkw = pk.at[w % 2]
                for slot in range(NSLOT):
                    unpack_from(pkw, slot, slot)
                    gather_start(slot)
                run_chunks(pkw, WIN)
                if w + 1 < nwin:
                    stage(w + 1).wait()

        plsc.subcore_barrier()
        if split_out:
            pltpu.sync_copy(acc.at[pl.ds(s * RPT, RPT), pl.ds(0, TD)],
                            out_hbm.at[c, pl.ds(s * RPT, RPT)])
            pltpu.sync_copy(acc.at[pl.ds(s * RPT, RPT), pl.ds(TD, D - TD)],
                            out2_hbm.at[c, pl.ds(s * RPT, RPT)])
        else:
            pltpu.sync_copy(acc.at[pl.ds(s * RPT, RPT)],
                            out_hbm.at[c, pl.ds(s * RPT, RPT)])

    return seg_sum


_seg_sum_d1 = _make_seg_sum(D1, IN_FEATS, True, NSLOT=3, WIN=25)
_seg_sum_d2 = _make_seg_sum(D2, D2, False, NSLOT=4)

_R = 2000


def _dense_body(x_ref, accf_ref, accd_ref, ws1_ref, wn1_ref, b1_ref,
                w2_ref, b2_ref, paug_ref, s_ref, rinv_ref):
    a = accf_ref[0] + accf_ref[1]
    ad = accd_ref[0] + accd_ref[1]
    deg = jnp.maximum(ad[:, 0:1], 1.0)
    hn = a / deg
    h = jnp.dot(x_ref[...], ws1_ref[...], preferred_element_type=jnp.float32)
    h = h + jnp.dot(hn, wn1_ref[...], preferred_element_type=jnp.float32)
    h = jnp.maximum(h + b1_ref[...], 0.0)
    ps = jnp.dot(h, w2_ref[...], preferred_element_type=jnp.float32)
    paug_ref[...] = ps[:, :D2]
    s_ref[...] = ps[:, D2:] + b2_ref[...]
    rinv_ref[...] = 1.0 / deg


def _pack_body(e_ref, out_ref):
    out_ref[...] = e_ref[0] | (e_ref[1] << 14)


def _epilogue_body(s_ref, acc2_ref, rinv_ref, out_ref):
    a2 = acc2_ref[0] + acc2_ref[1]
    out_ref[...] = (s_ref[...] + a2 * rinv_ref[...])[:, :2]


def kernel(inputs, edge_index, W_self1, W_neigh1, b1, W_self2, W_neigh2, b2):
    x = inputs
    ei = edge_index.astype(jnp.int32)
    pk = pl.pallas_call(
        _pack_body,
        grid=(1,),
        in_specs=[pl.BlockSpec((2, E), lambda i: (0, 0))],
        out_specs=pl.BlockSpec((E,), lambda i: (0,)),
        out_shape=jax.ShapeDtypeStruct((E,), jnp.int32),
    )(ei)

    xaug = jnp.concatenate(
        [x, jnp.ones((N, 1), x.dtype), jnp.zeros((N, D1 - IN_FEATS - 1),
                                                 x.dtype)], axis=1)
    accf, accd = _seg_sum_d1(xaug, pk)

    grid = (N // _R,)
    full = lambda shape: pl.BlockSpec(shape, lambda i: (0,) * len(shape))
    paug, s16, rinv = pl.pallas_call(
        _dense_body,
        grid=grid,
        in_specs=[
            pl.BlockSpec((_R, IN_FEATS), lambda i: (i, 0)),
            pl.BlockSpec((NC, _R, IN_FEATS), lambda i: (0, i, 0)),
            pl.BlockSpec((NC, _R, D1 - IN_FEATS), lambda i: (0, i, 0)),
            full((IN_FEATS, HIDDEN)),
            full((IN_FEATS, HIDDEN)),
            full((1, HIDDEN)),
            full((HIDDEN, 2 * D2)),
            full((1, D2)),
        ],
        out_specs=[
            pl.BlockSpec((_R, D2), lambda i: (i, 0)),
            pl.BlockSpec((_R, D2), lambda i: (i, 0)),
            pl.BlockSpec((_R, 1), lambda i: (i, 0)),
        ],
        out_shape=[
            jax.ShapeDtypeStruct((N, D2), jnp.float32),
            jax.ShapeDtypeStruct((N, D2), jnp.float32),
            jax.ShapeDtypeStruct((N, 1), jnp.float32),
        ],
    )(x, accf, accd, W_self1, W_neigh1, b1.reshape(1, HIDDEN),
      jnp.concatenate([jnp.pad(W_neigh2, ((0, 0), (0, D2 - 2))),
                       jnp.pad(W_self2, ((0, 0), (0, D2 - 2)))], axis=1),
      jnp.pad(b2, (0, D2 - 2)).reshape(1, D2))

    (acc2,) = _seg_sum_d2(paug, pk)

    out = pl.pallas_call(
        _epilogue_body,
        grid=grid,
        in_specs=[
            pl.BlockSpec((_R, D2), lambda i: (i, 0)),
            pl.BlockSpec((NC, _R, D2), lambda i: (0, i, 0)),
            pl.BlockSpec((_R, 1), lambda i: (i, 0)),
        ],
        out_specs=pl.BlockSpec((_R, 2), lambda i: (i, 0)),
        out_shape=jax.ShapeDtypeStruct((N, 2), jnp.float32),
    )(s16, acc2, rinv)
    return out


# speedup vs baseline: 1.4555x; 1.0355x over previous
import functools

import jax
import jax.numpy as jnp
from jax import lax
from jax.experimental import pallas as pl
from jax.experimental.pallas import tpu as pltpu
from jax.experimental.pallas import tpu_sc as plsc

N = 10000
E = 320000
IN_FEATS = 128
HIDDEN = 256

NC = 2
NS = 16
EPT = E // (NC * NS)
CHUNK = 80
NCH = EPT // CHUNK
ROWS = N
RPT = ROWS // NS
D1 = 144
D2 = 16


def _make_seg_sum(D, TD, split_out, NSLOT=2, WIN=None):
    d_lanes = D // 16
    mesh = plsc.VectorSubcoreMesh(core_axis_name="c", subcore_axis_name="s")
    if split_out:
        out_type = [jax.ShapeDtypeStruct((NC, ROWS, TD), jnp.float32),
                    jax.ShapeDtypeStruct((NC, ROWS, D - TD), jnp.float32)]
    else:
        out_type = [jax.ShapeDtypeStruct((NC, ROWS, D), jnp.float32)]

    @functools.partial(
        pl.kernel,
        mesh=mesh,
        compiler_params=pltpu.CompilerParams(use_tc_tiling_on_sc=False),
        out_type=out_type,
        scratch_types=[
            pltpu.VMEM((EPT,) if WIN is None else (2, WIN * CHUNK),
                       jnp.int32),
            pltpu.VMEM((NSLOT, CHUNK), jnp.int32),
            pltpu.VMEM((NSLOT, CHUNK), jnp.int32),
            pltpu.VMEM((NSLOT, CHUNK, D), jnp.float32),
            pltpu.VMEM_SHARED((ROWS, D), jnp.float32),
        ] + [pltpu.SemaphoreType.DMA] * (NSLOT + 1),
    )
    def seg_sum(table_hbm, pk_hbm, *out_and_scratch):
        if split_out:
            out_hbm, out2_hbm = out_and_scratch[:2]
            rest = out_and_scratch[2:]
        else:
            out_hbm, = out_and_scratch[:1]
            rest = out_and_scratch[1:]
        pk, srcb, dstb, rows, acc = rest[:5]
        sems = tuple(rest[5:5 + NSLOT])
        sem_pk = rest[5 + NSLOT]
        c = lax.axis_index("c")
        s = lax.axis_index("s")

        def zbody(t, carry):
            i = t // d_lanes
            j = t - i * d_lanes
            rows[0, i, pl.ds(j * 16, 16)] = jnp.zeros((16,), jnp.float32)
            return carry

        lax.fori_loop(0, CHUNK * d_lanes, zbody, 0)
        for z in range(RPT // CHUNK):
            pltpu.sync_copy(rows.at[0],
                            acc.at[pl.ds(s * RPT + z * CHUNK, CHUNK)])
        rem = RPT % CHUNK
        if rem:
            pltpu.sync_copy(
                rows.at[0, pl.ds(0, rem)],
                acc.at[pl.ds(s * RPT + (RPT // CHUNK) * CHUNK, rem)])

        base = (c * NS + s) * EPT

        def unpack_from(pkbuf, i, slot):
            for t in range(CHUNK // 16):
                v = pkbuf[pl.ds(i * CHUNK + t * 16, 16)]
                srcb[slot, pl.ds(t * 16, 16)] = v & 16383
                dstb[slot, pl.ds(t * 16, 16)] = lax.shift_right_logical(v, 14)

        def gather_start(slot):
            pltpu.make_async_copy(
                table_hbm.at[srcb.at[slot]], rows.at[slot], sems[slot]).start()

        def gather_wait(slot):
            pltpu.make_async_copy(
                table_hbm.at[srcb.at[slot]], rows.at[slot], sems[slot]).wait()

        def scatter(slot):
            pltpu.sync_copy(rows.at[slot], acc.at[dstb.at[slot]], add=True)

        def run_chunks(pkbuf, nch):
            def body(j, carry):
                for slot in range(NSLOT):
                    i = NSLOT * j + slot
                    gather_wait(slot)
                    scatter(slot)

                    @pl.when(i + NSLOT < nch)
                    def _():
                        unpack_from(pkbuf, i + NSLOT, slot)
                        gather_start(slot)

                return carry

            lax.fori_loop(0, nch // NSLOT, body, 0)
            for r in range(nch % NSLOT):
                gather_wait(r)
                scatter(r)

        if WIN is None:
            pltpu.sync_copy(pk_hbm.at[pl.ds(base, EPT)], pk)
            for slot in range(NSLOT):
                unpack_from(pk, slot, slot)
                gather_start(slot)
            plsc.subcore_barrier()
            run_chunks(pk, NCH)
        else:
            winw = WIN * CHUNK
            nwin = NCH // WIN

            def stage(w):
                return pltpu.make_async_copy(
                    pk_hbm.at[pl.ds(base + w * winw, winw)],
                    pk.at[w % 2], sem_pk)

            stage(0).start()
            plsc.subcore_barrier()
            stage(0).wait()
            for w in range(nwin):
                if w + 1 < nwin:
                    stage(w + 1).start()
                pkw = pk.at[w % 2]
                for slot in range(NSLOT):
                    unpack_from(pkw, slot, slot)
                    gather_start(slot)
                run_chunks(pkw, WIN)
                if w + 1 < nwin:
                    stage(w + 1).wait()

        plsc.subcore_barrier()
        if split_out:
            pltpu.sync_copy(acc.at[pl.ds(s * RPT, RPT), pl.ds(0, TD)],
                            out_hbm.at[c, pl.ds(s * RPT, RPT)])
            pltpu.sync_copy(acc.at[pl.ds(s * RPT, RPT), pl.ds(TD, D - TD)],
                            out2_hbm.at[c, pl.ds(s * RPT, RPT)])
        else:
            pltpu.sync_copy(acc.at[pl.ds(s * RPT, RPT)],
                            out_hbm.at[c, pl.ds(s * RPT, RPT)])

    return seg_sum


_seg_sum_d1 = _make_seg_sum(D1, IN_FEATS, True, NSLOT=3, WIN=25)
_seg_sum_d2 = _make_seg_sum(D2, D2, False, NSLOT=6)

_R = 2000


def _dense_body(x_ref, accf_ref, accd_ref, ws1_ref, wn1_ref, b1_ref,
                w2_ref, b2_ref, paug_ref, s_ref, rinv_ref):
    a = accf_ref[0] + accf_ref[1]
    ad = accd_ref[0] + accd_ref[1]
    deg = jnp.maximum(ad[:, 0:1], 1.0)
    hn = a / deg
    h = jnp.dot(x_ref[...], ws1_ref[...], preferred_element_type=jnp.float32)
    h = h + jnp.dot(hn, wn1_ref[...], preferred_element_type=jnp.float32)
    h = jnp.maximum(h + b1_ref[...], 0.0)
    ps = jnp.dot(h, w2_ref[...], preferred_element_type=jnp.float32)
    paug_ref[...] = ps[:, :D2]
    s_ref[...] = ps[:, D2:] + b2_ref[...]
    rinv_ref[...] = 1.0 / deg


def _pack_body(e_ref, out_ref):
    out_ref[...] = e_ref[0] | (e_ref[1] << 14)


def _epilogue_body(s_ref, acc2_ref, rinv_ref, out_ref):
    a2 = acc2_ref[0] + acc2_ref[1]
    out_ref[...] = (s_ref[...] + a2 * rinv_ref[...])[:, :2]


def kernel(inputs, edge_index, W_self1, W_neigh1, b1, W_self2, W_neigh2, b2):
    x = inputs
    ei = edge_index.astype(jnp.int32)
    pk = pl.pallas_call(
        _pack_body,
        grid=(1,),
        in_specs=[pl.BlockSpec((2, E), lambda i: (0, 0))],
        out_specs=pl.BlockSpec((E,), lambda i: (0,)),
        out_shape=jax.ShapeDtypeStruct((E,), jnp.int32),
    )(ei)

    xaug = jnp.concatenate(
        [x, jnp.ones((N, 1), x.dtype), jnp.zeros((N, D1 - IN_FEATS - 1),
                                                 x.dtype)], axis=1)
    accf, accd = _seg_sum_d1(xaug, pk)

    grid = (N // _R,)
    full = lambda shape: pl.BlockSpec(shape, lambda i: (0,) * len(shape))
    paug, s16, rinv = pl.pallas_call(
        _dense_body,
        grid=grid,
        in_specs=[
            pl.BlockSpec((_R, IN_FEATS), lambda i: (i, 0)),
            pl.BlockSpec((NC, _R, IN_FEATS), lambda i: (0, i, 0)),
            pl.BlockSpec((NC, _R, D1 - IN_FEATS), lambda i: (0, i, 0)),
            full((IN_FEATS, HIDDEN)),
            full((IN_FEATS, HIDDEN)),
            full((1, HIDDEN)),
            full((HIDDEN, 2 * D2)),
            full((1, D2)),
        ],
        out_specs=[
            pl.BlockSpec((_R, D2), lambda i: (i, 0)),
            pl.BlockSpec((_R, D2), lambda i: (i, 0)),
            pl.BlockSpec((_R, 1), lambda i: (i, 0)),
        ],
        out_shape=[
            jax.ShapeDtypeStruct((N, D2), jnp.float32),
            jax.ShapeDtypeStruct((N, D2), jnp.float32),
            jax.ShapeDtypeStruct((N, 1), jnp.float32),
        ],
    )(x, accf, accd, W_self1, W_neigh1, b1.reshape(1, HIDDEN),
      jnp.concatenate([jnp.pad(W_neigh2, ((0, 0), (0, D2 - 2))),
                       jnp.pad(W_self2, ((0, 0), (0, D2 - 2)))], axis=1),
      jnp.pad(b2, (0, D2 - 2)).reshape(1, D2))

    (acc2,) = _seg_sum_d2(paug, pk)

    out = pl.pallas_call(
        _epilogue_body,
        grid=grid,
        in_specs=[
            pl.BlockSpec((_R, D2), lambda i: (i, 0)),
            pl.BlockSpec((NC, _R, D2), lambda i: (0, i, 0)),
            pl.BlockSpec((_R, 1), lambda i: (i, 0)),
        ],
        out_specs=pl.BlockSpec((_R, 2), lambda i: (i, 0)),
        out_shape=jax.ShapeDtypeStruct((N, 2), jnp.float32),
    )(s16, acc2, rinv)
    return out
